# baseline (speedup 1.0000x reference)
import numpy as np

import jax
import jax.numpy as jnp
from jax.experimental import pallas as pl
from jax.experimental.pallas import tpu as pltpu

N = 10000
NFEAT = 128
NHID = 16
NCLASS = 8
NC2 = NHID + NCLASS

B = 8
BT = 1280
NPAD = B * BT
VC = N - (B - 1) * BT
BH = BT // 2
VCR = VC - BH


def _build_schedule():
    rows = []
    seen_out = set()
    for r in range(B):
        for k in range(B):
            do_out = 1 if k < r else 0
            of = 1 if (do_out and r not in seen_out) else 0
            if of:
                seen_out.add(r)
            rows.append([r, k, k if r == 0 else -1, 1 if r == 0 else 0,
                         1, do_out, 1 if k == B - 1 else 0,
                         1 if k == 0 else 0, of, r])
        for a2 in range(r + 1):
            of = 1 if a2 not in seen_out else 0
            if of:
                seen_out.add(a2)
            rows.append([a2, r, -1, 0, 0, 1, 0, 0, of, r])
    arr = np.asarray(rows, dtype=np.int32).T
    xi = arr[2]
    last = 0
    for i in range(xi.shape[0]):
        if xi[i] < 0:
            xi[i] = last
        else:
            last = xi[i]
    return arr


_SCHED = _build_schedule()
_S = _SCHED.shape[1]


def _body(sched, adjl_ref, adjr_ref, x_ref, w1_ref, b1_ref, w2_ref, b2_ref,
          h_out, emb2_out, logp_out, s12_scr, out_scr):
    i = pl.program_id(0)
    a = sched[0, i]
    c = sched[1, i]
    do_s1 = sched[3, i]
    do_h = sched[4, i]
    do_out = sched[5, i]
    do_s2 = sched[6, i]
    h_first = sched[7, i]
    out_first = sched[8, i]

    @pl.when(do_s1 == 1)
    def _():
        s1v = jnp.dot(x_ref[...], w1_ref[...],
                      preferred_element_type=jnp.float32)
        rowid = c * BT + jax.lax.broadcasted_iota(jnp.int32, (BT, NHID), 0)
        s12_scr[pl.ds(c * BT, BT), 0:NHID] = jnp.where(rowid < N, s1v, 0.0)

    def _updates(res):
        @pl.when(do_h == 1)
        def _():
            prev = jnp.where(h_first == 1,
                             jnp.broadcast_to(b1_ref[...], (BT, NHID)),
                             h_out[...])
            h_out[...] = prev + res[:, 0:NHID]

        @pl.when(do_s2 == 1)
        def _():
            s2v = jnp.dot(jax.nn.relu(h_out[...]), w2_ref[...],
                          preferred_element_type=jnp.float32)
            rowid = a * BT + jax.lax.broadcasted_iota(
                jnp.int32, (BT, NCLASS), 0)
            s12_scr[pl.ds(a * BT, BT), NHID:NC2] = jnp.where(
                rowid < N, s2v, 0.0)

        @pl.when(do_out == 1)
        def _():
            prev = jnp.where(out_first == 1,
                             jnp.broadcast_to(b2_ref[...], (BT, NCLASS)),
                             out_scr[pl.ds(a * BT, BT), :])
            out_scr[pl.ds(a * BT, BT), :] = prev + res[:, NHID:NC2]

    @pl.when(c < B - 1)
    def _():
        _updates(
            jnp.dot(adjl_ref[...], s12_scr[pl.ds(c * BT, BH), :],
                    preferred_element_type=jnp.float32)
            + jnp.dot(adjr_ref[...], s12_scr[pl.ds(c * BT + BH, BH), :],
                      preferred_element_type=jnp.float32))

    @pl.when(c == B - 1)
    def _():
        _updates(
            jnp.dot(adjl_ref[...], s12_scr[pl.ds(c * BT, BH), :],
                    preferred_element_type=jnp.float32)
            + jnp.dot(adjr_ref[:, 0:VCR], s12_scr[pl.ds(c * BT + BH, VCR), :],
                      preferred_element_type=jnp.float32))

    @pl.when(i == _S - 1)
    def _():
        o = out_scr[0:N, :]
        emb2_out[...] = o
        m = jnp.max(o, axis=1, keepdims=True)
        z = o - m
        logp_out[...] = z - jnp.log(jnp.sum(jnp.exp(z), axis=1,
                                            keepdims=True))


@jax.jit
def kernel(x, adj, W1, b1, W2, b2):
    b1r = b1.reshape(1, NHID)
    b2r = b2.reshape(1, NCLASS)
    sched = jnp.asarray(_SCHED)

    grid_spec = pltpu.PrefetchScalarGridSpec(
        num_scalar_prefetch=1,
        grid=(_S,),
        in_specs=[
            pl.BlockSpec((BT, BH), lambda i, s: (s[0, i], 2 * s[1, i])),
            pl.BlockSpec((BT, BH), lambda i, s: (s[0, i], 2 * s[1, i] + 1)),
            pl.BlockSpec((BT, NFEAT), lambda i, s: (s[2, i], 0)),
            pl.BlockSpec((NFEAT, NHID), lambda i, s: (0, 0)),
            pl.BlockSpec((1, NHID), lambda i, s: (0, 0)),
            pl.BlockSpec((NHID, NCLASS), lambda i, s: (0, 0)),
            pl.BlockSpec((1, NCLASS), lambda i, s: (0, 0)),
        ],
        out_specs=[
            pl.BlockSpec((BT, NHID), lambda i, s: (s[9, i], 0)),
            pl.BlockSpec((N, NCLASS), lambda i, s: (0, 0)),
            pl.BlockSpec((N, NCLASS), lambda i, s: (0, 0)),
        ],
        scratch_shapes=[
            pltpu.VMEM((NPAD, NC2), jnp.float32),
            pltpu.VMEM((NPAD, NCLASS), jnp.float32),
        ],
    )

    h, emb2, logp = pl.pallas_call(
        _body,
        grid_spec=grid_spec,
        out_shape=[
            jax.ShapeDtypeStruct((N, NHID), jnp.float32),
            jax.ShapeDtypeStruct((N, NCLASS), jnp.float32),
            jax.ShapeDtypeStruct((N, NCLASS), jnp.float32),
        ],
    )(sched, adj, adj, x, W1, b1r, W2, b2r)

    return (logp, h, emb2)


# speedup vs baseline: 1.1312x; 1.1312x over previous
import numpy as np

import jax
import jax.numpy as jnp
from jax.experimental import pallas as pl
from jax.experimental.pallas import tpu as pltpu

N = 10000
NFEAT = 128
NHID = 16
NCLASS = 8
NC2 = NHID + NCLASS

B = 5
BT = 2048
NPAD = B * BT
VC = N - (B - 1) * BT
BH = BT // 2
VCR = VC - BH


def _build_schedule():
    rows = []
    seen_out = set()
    for r in range(B):
        for k in range(B):
            do_out = 1 if k < r else 0
            of = 1 if (do_out and r not in seen_out) else 0
            if of:
                seen_out.add(r)
            rows.append([r, k, k if r == 0 else -1, 1 if r == 0 else 0,
                         1, do_out, 1 if k == B - 1 else 0,
                         1 if k == 0 else 0, of, r, 0, 0])
        for a2 in range(r + 1):
            of = 1 if a2 not in seen_out else 0
            if of:
                seen_out.add(a2)
            fin = 1 if r == B - 1 else 0
            rows.append([a2, r, -1, 0, 0, 1, 0, 0, of, r,
                         a2 if fin else 0, fin])
    arr = np.asarray(rows, dtype=np.int32).T
    xi = arr[2]
    last = 0
    for i in range(xi.shape[0]):
        if xi[i] < 0:
            xi[i] = last
        else:
            last = xi[i]
    return arr


_SCHED = _build_schedule()
_S = _SCHED.shape[1]


def _body(sched, adjl_ref, adjr_ref, x_ref, w1_ref, b1_ref, w2_ref, b2_ref,
          h_out, emb2_out, logp_out, s12_scr, out_scr):
    i = pl.program_id(0)
    a = sched[0, i]
    c = sched[1, i]
    do_s1 = sched[3, i]
    do_h = sched[4, i]
    do_out = sched[5, i]
    do_s2 = sched[6, i]
    h_first = sched[7, i]
    out_first = sched[8, i]
    fin = sched[11, i]

    @pl.when(do_s1 == 1)
    def _():
        s1v = jnp.dot(x_ref[...], w1_ref[...],
                      preferred_element_type=jnp.float32)
        rowid = c * BT + jax.lax.broadcasted_iota(jnp.int32, (BT, NHID), 0)
        s12_scr[pl.ds(c * BT, BT), 0:NHID] = jnp.where(rowid < N, s1v, 0.0)

    def _updates(res):
        @pl.when(do_h == 1)
        def _():
            prev = jnp.where(h_first == 1,
                             jnp.broadcast_to(b1_ref[...], (BT, NHID)),
                             h_out[...])
            h_out[...] = prev + res[:, 0:NHID]

        @pl.when(do_s2 == 1)
        def _():
            s2v = jnp.dot(jax.nn.relu(h_out[...]), w2_ref[...],
                          preferred_element_type=jnp.float32)
            rowid = a * BT + jax.lax.broadcasted_iota(
                jnp.int32, (BT, NCLASS), 0)
            s12_scr[pl.ds(a * BT, BT), NHID:NC2] = jnp.where(
                rowid < N, s2v, 0.0)

        @pl.when(do_out == 1)
        def _():
            prev = jnp.where(out_first == 1,
                             jnp.broadcast_to(b2_ref[...], (BT, NCLASS)),
                             out_scr[pl.ds(a * BT, BT), :])
            out_scr[pl.ds(a * BT, BT), :] = prev + res[:, NHID:NC2]

    @pl.when(c < B - 1)
    def _():
        _updates(
            jnp.dot(adjl_ref[...], s12_scr[pl.ds(c * BT, BH), :],
                    preferred_element_type=jnp.float32)
            + jnp.dot(adjr_ref[...], s12_scr[pl.ds(c * BT + BH, BH), :],
                      preferred_element_type=jnp.float32))

    @pl.when(c == B - 1)
    def _():
        _updates(
            jnp.dot(adjl_ref[...], s12_scr[pl.ds(c * BT, BH), :],
                    preferred_element_type=jnp.float32)
            + jnp.dot(adjr_ref[:, 0:VCR], s12_scr[pl.ds(c * BT + BH, VCR), :],
                      preferred_element_type=jnp.float32))

    @pl.when(fin == 1)
    def _():
        o = out_scr[pl.ds(a * BT, BT), :]
        emb2_out[...] = o
        m = jnp.max(o, axis=1, keepdims=True)
        z = o - m
        logp_out[...] = z - jnp.log(jnp.sum(jnp.exp(z), axis=1,
                                            keepdims=True))


@jax.jit
def kernel(x, adj, W1, b1, W2, b2):
    b1r = b1.reshape(1, NHID)
    b2r = b2.reshape(1, NCLASS)
    sched = jnp.asarray(_SCHED)

    grid_spec = pltpu.PrefetchScalarGridSpec(
        num_scalar_prefetch=1,
        grid=(_S,),
        in_specs=[
            pl.BlockSpec((BT, BH), lambda i, s: (s[0, i], 2 * s[1, i])),
            pl.BlockSpec((BT, BH), lambda i, s: (s[0, i], 2 * s[1, i] + 1)),
            pl.BlockSpec((BT, NFEAT), lambda i, s: (s[2, i], 0)),
            pl.BlockSpec((NFEAT, NHID), lambda i, s: (0, 0)),
            pl.BlockSpec((1, NHID), lambda i, s: (0, 0)),
            pl.BlockSpec((NHID, NCLASS), lambda i, s: (0, 0)),
            pl.BlockSpec((1, NCLASS), lambda i, s: (0, 0)),
        ],
        out_specs=[
            pl.BlockSpec((BT, NHID), lambda i, s: (s[9, i], 0)),
            pl.BlockSpec((BT, NCLASS), lambda i, s: (s[10, i], 0)),
            pl.BlockSpec((BT, NCLASS), lambda i, s: (s[10, i], 0)),
        ],
        scratch_shapes=[
            pltpu.VMEM((NPAD, NC2), jnp.float32),
            pltpu.VMEM((NPAD, NCLASS), jnp.float32),
        ],
    )

    h, emb2, logp = pl.pallas_call(
        _body,
        grid_spec=grid_spec,
        out_shape=[
            jax.ShapeDtypeStruct((N, NHID), jnp.float32),
            jax.ShapeDtypeStruct((N, NCLASS), jnp.float32),
            jax.ShapeDtypeStruct((N, NCLASS), jnp.float32),
        ],
    )(sched, adj, adj, x, W1, b1r, W2, b2r)

    return (logp, h, emb2)
